# baseline (speedup 1.0000x reference)
import functools

import jax
import jax.numpy as jnp
from jax import lax
from jax.experimental import pallas as pl
from jax.experimental.pallas import tpu as pltpu
from jax.experimental.pallas import tpu_sc as plsc

TEMPERATURE = 0.1
N_NEG = 10
EPS = 1e-8

B, D, H, W = 8, 512, 8, 128
T = H * W
R = B * T
NS = T // 128
NSEG = NS // 2
LANES = 16
GPR = 128 // LANES
NW = 32
TOK_W = R // NW
NEG_W = TOK_W * N_NEG
CH = 64
NCH = TOK_W // CH
IDX_ROWS = TOK_W * LANES // 128
PACKED_ROWS = R * LANES // 128
NEG_INF = float("-inf")


def _gram_body(o_ref, p_ref, out_ref):
    o = o_ref[0]
    p = p_ref[0]
    no = jnp.maximum(jnp.sqrt(jnp.sum(o * o, axis=0, keepdims=True)), EPS)
    on = ((o * ((1.0 / TEMPERATURE) / no)).astype(jnp.bfloat16)).T
    npv = jnp.maximum(jnp.sqrt(jnp.sum(p * p, axis=0, keepdims=True)), EPS)
    pn = (p * (1.0 / npv)).astype(jnp.bfloat16)
    d = lax.dot_general(on, pn, (((1,), (0,)), ((), ())),
                        preferred_element_type=jnp.float32)
    for cs2 in range(NSEG):
        lo16 = lax.convert_element_type(
            lax.bitcast_convert_type(
                d[:, cs2 * 256:cs2 * 256 + 128].astype(jnp.bfloat16),
                jnp.uint16), jnp.int32)
        hi16 = lax.convert_element_type(
            lax.bitcast_convert_type(
                d[:, cs2 * 256 + 128:cs2 * 256 + 256].astype(jnp.bfloat16),
                jnp.uint16), jnp.int32)
        out_ref[pl.ds(cs2 * T, T), :] = lo16 | lax.shift_left(hi16, 16)


def _gram(orig_r, pred_r):
    return pl.pallas_call(
        _gram_body,
        grid=(B,),
        in_specs=[
            pl.BlockSpec((1, D, T), lambda b: (b, 0, 0)),
            pl.BlockSpec((1, D, T), lambda b: (b, 0, 0)),
        ],
        out_specs=pl.BlockSpec((NSEG * T, 128), lambda b: (b, 0)),
        out_shape=jax.ShapeDtypeStruct((B * NSEG * T, 128), jnp.int32),
    )(orig_r, pred_r)


@functools.partial(
    pl.kernel,
    mesh=plsc.VectorSubcoreMesh(core_axis_name="c", subcore_axis_name="s"),
    out_type=jax.ShapeDtypeStruct((PACKED_ROWS, 128), jnp.float32),
    compiler_params=pltpu.CompilerParams(
        use_tc_tiling_on_sc=False, needs_layout_passes=False),
    scratch_types=[
        pltpu.VMEM((NEG_W + LANES,), jnp.int32),
        pltpu.VMEM((NSEG * CH, 128), jnp.int32),
        pltpu.VMEM((NSEG * CH, 128), jnp.int32),
        pltpu.VMEM((IDX_ROWS, 128), jnp.float32),
        pltpu.SemaphoreType.DMA,
        pltpu.SemaphoreType.DMA,
    ],
)
def _sc_gather(table_hbm, neg_hbm, out_hbm, idx_v, rows_a, rows_b, out_v,
               sem_a, sem_b):
    wid = lax.axis_index("s") * 2 + lax.axis_index("c")
    b = wid // (NW // B)
    t0 = (wid % (NW // B)) * TOK_W
    pltpu.sync_copy(neg_hbm.at[pl.ds(wid * NEG_W, NEG_W)],
                    idx_v.at[pl.ds(0, NEG_W)])

    bufs = (rows_a, rows_b)
    sems = (sem_a, sem_b)

    def fire(c):
        buf = bufs[c % 2]
        sem = sems[c % 2]
        return [
            pltpu.async_copy(
                table_hbm.at[pl.ds((b * NSEG + sg) * T + t0 + c * CH, CH)],
                buf.at[pl.ds(sg * CH, CH)], sem)
            for sg in range(NSEG)
        ]

    lane_i = lax.iota(jnp.int32, LANES)
    pending = fire(0)
    for c in range(NCH):
        nxt = fire(c + 1) if c + 1 < NCH else []
        for cp in pending:
            cp.wait()
        pending = nxt
        buf = bufs[c % 2]

        def body(i, carry, c=c, buf=buf):
            k = c * CH + i
            raw = idx_v[pl.ds(k * N_NEG, LANES)]
            neg_col = lax.bitwise_or(
                lax.shift_left(lax.bitwise_and(raw, H - 1), 7),
                lax.shift_right_logical(raw, 3))
            pos = jnp.zeros((LANES,), jnp.int32) + (t0 + k)
            col = jnp.where(lane_i >= N_NEG, pos, neg_col)
            row = lax.shift_right_logical(col, 8) * CH + i
            lane = lax.bitwise_and(col, 127)
            w = plsc.load_gather(buf, [row, lane])
            half = lax.bitwise_and(lax.shift_right_logical(col, 7), 1)
            bits = lax.bitwise_and(
                lax.shift_right_logical(w, half * 16), 0xFFFF)
            val = plsc.bitcast(lax.shift_left(bits, 16), jnp.float32)
            dead = (lane_i > N_NEG) | ((lane_i < N_NEG) & (col == pos))
            val = jnp.where(dead, jnp.float32(NEG_INF), val)
            out_v[k // GPR, pl.ds((k % GPR) * LANES, LANES)] = val
            return carry

        lax.fori_loop(0, CH, body, 0)
    pltpu.sync_copy(out_v, out_hbm.at[pl.ds(wid * IDX_ROWS, IDX_ROWS)])


def _finish_body(vals_ref, out_ref):
    vals = vals_ref[...]
    lane = lax.broadcasted_iota(jnp.int32, (PACKED_ROWS, 128), 1)
    sub = lane % LANES
    e = jnp.exp(vals)
    li = lax.broadcasted_iota(jnp.int32, (128, GPR), 0)
    gi = lax.broadcasted_iota(jnp.int32, (128, GPR), 1)
    sel = (li // LANES == gi).astype(jnp.float32)
    dims = (((1,), (0,)), ((), ()))
    negsum = lax.dot_general(jnp.where(sub < N_NEG, e, 0.0), sel, dims,
                             preferred_element_type=jnp.float32)
    epos = lax.dot_general(jnp.where(sub == N_NEG, e, 0.0), sel, dims,
                           preferred_element_type=jnp.float32)
    pos = lax.dot_general(jnp.where(sub == N_NEG, vals, 0.0), sel, dims,
                          preferred_element_type=jnp.float32)
    lse = jnp.log(epos + negsum)
    out_ref[...] = jnp.sum(lse - pos, keepdims=True) * (1.0 / R)


def _finish(vals_p):
    return pl.pallas_call(
        _finish_body,
        out_shape=jax.ShapeDtypeStruct((1, 1), jnp.float32),
    )(vals_p)


def kernel(pred_tokens, original_tokens):
    ghat = _gram(original_tokens.reshape(B, D, T), pred_tokens.reshape(B, D, T))

    neg_inds = jax.random.randint(
        jax.random.key(42), (R * N_NEG,), 0, T - 1).astype(jnp.int32)

    gathered = _sc_gather(ghat, neg_inds)
    return _finish(gathered).reshape(())
